# Initial kernel scaffold; baseline (speedup 1.0000x reference)
import functools

import jax
import jax.numpy as jnp
from jax import lax
from jax.experimental import pallas as pl
from jax.experimental.pallas import tpu as pltpu
from jax.experimental.pallas import tpu_sc as plsc

B = 1024
L = 200
D = 1024
SUB = 2
NTOK = L // SUB
TT = B * NTOK
NW = 32
TOK_PER_W = TT // NW
T = 32
CHUNKS = TOK_PER_W // T
LANES = 16
DV = D // LANES


def _body(idx_hbm, table_hbm, out_hbm, idx_v, rows_v, out_v, sem):
    wid = lax.axis_index("s") * 2 + lax.axis_index("c")
    pltpu.sync_copy(idx_hbm.at[pl.ds(wid * CHUNKS, CHUNKS)], idx_v)

    def step(g, carry):
        pltpu.async_copy(table_hbm.at[idx_v.at[g]], rows_v, sem).wait()

        def tok(t, c2):
            def lane(j, c3):
                s = pl.ds(j * LANES, LANES)
                a = rows_v[2 * t, s]
                b = rows_v[2 * t + 1, s]
                out_v[t, s] = (a + b) * 0.5
                return c3
            return lax.fori_loop(0, DV, lane, c2, unroll=4)

        lax.fori_loop(0, T, tok, carry)
        pltpu.sync_copy(out_v, out_hbm.at[pl.ds(wid * TOK_PER_W + g * T, T)])
        return carry

    lax.fori_loop(0, CHUNKS, step, 0)


_gather_mean = functools.partial(
    pl.kernel,
    out_type=jax.ShapeDtypeStruct((TT, D), jnp.float32),
    mesh=plsc.VectorSubcoreMesh(core_axis_name="c", subcore_axis_name="s"),
    scratch_types=[
        pltpu.VMEM((CHUNKS, 2 * T), jnp.int32),
        pltpu.VMEM((2 * T, D), jnp.float32),
        pltpu.VMEM((T, D), jnp.float32),
        pltpu.SemaphoreType.DMA,
    ],
)(_body)


def kernel(tokens_ids, seg_ids, emb_table):
    idx = tokens_ids.reshape(NW * CHUNKS, 2 * T)
    out = _gather_mean(idx, emb_table)
    return out.reshape(B, NTOK, D)


# speedup vs baseline: 1.2476x; 1.2476x over previous
import functools

import jax
import jax.numpy as jnp
from jax import lax
from jax.experimental import pallas as pl
from jax.experimental.pallas import tpu as pltpu
from jax.experimental.pallas import tpu_sc as plsc

B = 1024
L = 200
D = 1024
SUB = 2
NTOK = L // SUB
TT = B * NTOK
NW = 32
TOK_PER_W = TT // NW
T = 32
CHUNKS = TOK_PER_W // T
LANES = 16
DV = D // LANES


def _body(idx_hbm, table_hbm, out_hbm, idx_v, rows_v, out_v, sem):
    wid = lax.axis_index("s") * 2 + lax.axis_index("c")
    pltpu.sync_copy(idx_hbm.at[wid], idx_v)

    def step(g, carry):
        pltpu.async_copy(table_hbm.at[idx_v.at[g]], rows_v, sem).wait()

        def tok(t, c2):
            def lane(j, c3):
                s = pl.ds(j * LANES, LANES)
                a = rows_v[2 * t, s]
                b = rows_v[2 * t + 1, s]
                out_v[t, s] = (a + b) * 0.5
                return c3
            return lax.fori_loop(0, DV, lane, c2, unroll=4)

        lax.fori_loop(0, T, tok, carry)
        pltpu.sync_copy(out_v, out_hbm.at[pl.ds(wid * TOK_PER_W + g * T, T)])
        return carry

    lax.fori_loop(0, CHUNKS, step, 0)


_gather_mean = functools.partial(
    pl.kernel,
    out_type=jax.ShapeDtypeStruct((TT, D), jnp.float32),
    mesh=plsc.VectorSubcoreMesh(core_axis_name="c", subcore_axis_name="s"),
    scratch_types=[
        pltpu.VMEM((CHUNKS, 2 * T), jnp.int32),
        pltpu.VMEM((2 * T, D), jnp.float32),
        pltpu.VMEM((T, D), jnp.float32),
        pltpu.SemaphoreType.DMA,
    ],
)(_body)


def kernel(tokens_ids, seg_ids, emb_table):
    idx = tokens_ids.reshape(NW, CHUNKS, 2 * T)
    out = _gather_mean(idx, emb_table)
    return out.reshape(B, NTOK, D)


# speedup vs baseline: 1.5870x; 1.2721x over previous
import functools

import jax
import jax.numpy as jnp
from jax import lax
from jax.experimental import pallas as pl
from jax.experimental.pallas import tpu as pltpu
from jax.experimental.pallas import tpu_sc as plsc

B = 1024
L = 200
D = 1024
SUB = 2
NTOK = L // SUB
TT = B * NTOK
NW = 32
TOK_PER_W = TT // NW
T = 16
CHUNKS = TOK_PER_W // T
PAIRS = CHUNKS // 2
LANES = 16
DV = D // LANES


def _body(idx_hbm, table_hbm, out_hbm,
          idx_v, rows0, rows1, out0, out1, sg0, sg1, so0, so1):
    wid = lax.axis_index("s") * 2 + lax.axis_index("c")
    pltpu.sync_copy(idx_hbm.at[wid], idx_v)
    rows = (rows0, rows1)
    outs = (out0, out1)
    sgs = (sg0, sg1)
    sos = (so0, so1)

    def start_gather(g, b):
        pltpu.async_copy(table_hbm.at[idx_v.at[g]], rows[b], sgs[b])

    def wait_gather(b):
        pltpu.make_async_copy(table_hbm.at[idx_v.at[0]], rows[b], sgs[b]).wait()

    def start_out(g, b):
        pltpu.async_copy(
            outs[b], out_hbm.at[pl.ds(wid * TOK_PER_W + g * T, T)], sos[b])

    def wait_out(b):
        pltpu.make_async_copy(
            outs[b], out_hbm.at[pl.ds(0, T)], sos[b]).wait()

    def compute(b):
        rv, ov = rows[b], outs[b]

        def tok(t, c):
            def lane(j, c3):
                s = pl.ds(j * LANES, LANES)
                ov[t, s] = (rv[2 * t, s] + rv[2 * t + 1, s]) * 0.5
                return c3
            return lax.fori_loop(0, DV, lane, c, unroll=8)

        lax.fori_loop(0, T, tok, 0)

    start_gather(0, 0)

    def pairstep(p, carry):
        g0 = 2 * p
        wait_gather(0)
        start_gather(g0 + 1, 1)

        @pl.when(p > 0)
        def _():
            wait_out(0)
        compute(0)
        start_out(g0, 0)

        wait_gather(1)

        @pl.when(p < PAIRS - 1)
        def _():
            start_gather(g0 + 2, 0)

        @pl.when(p > 0)
        def _():
            wait_out(1)
        compute(1)
        start_out(g0 + 1, 1)
        return carry

    lax.fori_loop(0, PAIRS, pairstep, 0)
    wait_out(0)
    wait_out(1)


_gather_mean = functools.partial(
    pl.kernel,
    out_type=jax.ShapeDtypeStruct((TT, D), jnp.float32),
    mesh=plsc.VectorSubcoreMesh(core_axis_name="c", subcore_axis_name="s"),
    scratch_types=[
        pltpu.VMEM((CHUNKS, 2 * T), jnp.int32),
        pltpu.VMEM((2 * T, D), jnp.float32),
        pltpu.VMEM((2 * T, D), jnp.float32),
        pltpu.VMEM((T, D), jnp.float32),
        pltpu.VMEM((T, D), jnp.float32),
        pltpu.SemaphoreType.DMA,
        pltpu.SemaphoreType.DMA,
        pltpu.SemaphoreType.DMA,
        pltpu.SemaphoreType.DMA,
    ],
)(_body)


def kernel(tokens_ids, seg_ids, emb_table):
    idx = tokens_ids.reshape(NW, CHUNKS, 2 * T)
    out = _gather_mean(idx, emb_table)
    return out.reshape(B, NTOK, D)


# speedup vs baseline: 3.7135x; 2.3399x over previous
import functools

import jax
import jax.numpy as jnp
from jax import lax
from jax.experimental import pallas as pl
from jax.experimental.pallas import tpu as pltpu
from jax.experimental.pallas import tpu_sc as plsc

B = 1024
L = 200
D = 1024
SUB = 2
NTOK = L // SUB
NCG = 8
NTW = 4
CW = D // NCG
NBLK = 4
SPB = B // NTW // NBLK
LANES = 16
VPT = CW // LANES


def _body(idx_hbm, table_hbm, out_hbm,
          idx_v, rows0, rows1, out0, out1, sg0, sg1, so0, so1):
    wid = lax.axis_index("s") * 2 + lax.axis_index("c")
    gc = wid % NCG
    tw = wid // NCG
    col = pl.multiple_of(gc * CW, CW)
    rows = (rows0, rows1)
    outs = (out0, out1)
    sgs = (sg0, sg1)
    sos = (so0, so1)

    def start_gather(i, par):
        for j in range(2):
            pltpu.async_copy(
                table_hbm.at[idx_v.at[i, j], pl.ds(col, CW)],
                rows[par].at[pl.ds(j * NTOK, NTOK)], sgs[par])

    def wait_gather(par):
        for _ in range(2):
            pltpu.make_async_copy(
                table_hbm.at[idx_v.at[0, 0], pl.ds(col, CW)],
                rows[par].at[pl.ds(0, NTOK)], sgs[par]).wait()

    def start_out(s, par):
        pltpu.async_copy(outs[par], out_hbm.at[s, :, pl.ds(col, CW)], sos[par])

    def wait_out(par):
        pltpu.make_async_copy(
            outs[par], out_hbm.at[0, :, pl.ds(col, CW)], sos[par]).wait()

    def compute(par):
        rv, ov = rows[par], outs[par]

        @plsc.parallel_loop(0, NTOK * VPT, unroll=8)
        def _(k):
            t = k // VPT
            j = k % VPT
            s = pl.ds(j * LANES, LANES)
            ov[t, s] = (rv[2 * t, s] + rv[2 * t + 1, s]) * 0.5

    for blk in range(NBLK):
        pltpu.sync_copy(idx_hbm.at[tw, blk], idx_v)
        start_gather(0, 0)

        def pairstep(p, carry, blk=blk):
            sbase = tw * (SPB * NBLK) + blk * SPB
            for half in range(2):
                i = 2 * p + half
                par = half
                wait_gather(par)
                if half == 0:
                    start_gather(i + 1, 1 - par)
                else:
                    @pl.when(p < SPB // 2 - 1)
                    def _():
                        start_gather(i + 1, 1 - par)
                if blk == 0:
                    @pl.when(p > 0)
                    def _():
                        wait_out(par)
                else:
                    wait_out(par)
                compute(par)
                start_out(sbase + i, par)
            return carry

        lax.fori_loop(0, SPB // 2, pairstep, 0)
    wait_out(0)
    wait_out(1)


_gather_mean = functools.partial(
    pl.kernel,
    out_type=jax.ShapeDtypeStruct((B, NTOK, D), jnp.float32),
    mesh=plsc.VectorSubcoreMesh(core_axis_name="c", subcore_axis_name="s"),
    scratch_types=[
        pltpu.VMEM((SPB, SUB, NTOK), jnp.int32),
        pltpu.VMEM((L, CW), jnp.float32),
        pltpu.VMEM((L, CW), jnp.float32),
        pltpu.VMEM((NTOK, CW), jnp.float32),
        pltpu.VMEM((NTOK, CW), jnp.float32),
        pltpu.SemaphoreType.DMA,
        pltpu.SemaphoreType.DMA,
        pltpu.SemaphoreType.DMA,
        pltpu.SemaphoreType.DMA,
    ],
)(_body)


def kernel(tokens_ids, seg_ids, emb_table):
    idx = tokens_ids.reshape(NTW, NBLK, SPB, SUB, NTOK)
    return _gather_mean(idx, emb_table)


# speedup vs baseline: 5.8839x; 1.5845x over previous
import functools

import jax
import jax.numpy as jnp
from jax import lax
from jax.experimental import pallas as pl
from jax.experimental.pallas import tpu as pltpu
from jax.experimental.pallas import tpu_sc as plsc

B = 1024
L = 200
D = 1024
SUB = 2
NTOK = L // SUB
NCG = 8
NBW = 4
CW = D // NCG
BB = 128
NBB = B // BB
TB = 50
LANES = 16
VPT = CW // LANES


def _body(idx_hbm, table_hbm, out_hbm,
          idx_v, rows0, rows1, out0, out1, sg0, sg1, so0, so1):
    wid = lax.axis_index("s") * 2 + lax.axis_index("c")
    gc = wid % NCG
    bw = wid // NCG
    col = pl.multiple_of(gc * CW, CW)
    rows = (rows0, rows1)
    outs = (out0, out1)
    sgs = (sg0, sg1)
    sos = (so0, so1)

    def start_gather(tp, par):
        for j in range(SUB):
            pltpu.async_copy(
                table_hbm.at[idx_v.at[tp, pl.ds(j * BB, BB)], pl.ds(col, CW)],
                rows[par].at[pl.ds(j * BB, BB)], sgs[par])

    def wait_gather(par):
        for j in range(SUB):
            pltpu.make_async_copy(
                table_hbm.at[idx_v.at[0, pl.ds(0, BB)], pl.ds(col, CW)],
                rows[par].at[pl.ds(0, BB)], sgs[par]).wait()

    def start_out(t, bb, par):
        b0 = pl.multiple_of(bb * BB, BB)
        pltpu.async_copy(
            outs[par], out_hbm.at[t, pl.ds(b0, BB), pl.ds(col, CW)], sos[par])

    def wait_out(par):
        pltpu.make_async_copy(
            outs[par], out_hbm.at[0, pl.ds(0, BB), pl.ds(col, CW)],
            sos[par]).wait()

    def compute(par):
        rv, ov = rows[par], outs[par]

        @plsc.parallel_loop(0, BB, unroll=2)
        def _(q):
            for j in range(VPT):
                s = pl.ds(j * LANES, LANES)
                ov[q, s] = (rv[q, s] + rv[BB + q, s]) * 0.5

    def blkstep(blk, carry):
        bb = bw * 2 + (blk >> 1)
        h = blk & 1
        pltpu.sync_copy(idx_hbm.at[bb, h], idx_v)
        start_gather(0, 0)

        def pairstep(p, c):
            for half in range(2):
                tp = 2 * p + half
                par = half
                wait_gather(par)
                if half == 0:
                    start_gather(tp + 1, 1 - par)
                else:
                    @pl.when(p < TB // 2 - 1)
                    def _():
                        start_gather(tp + 1, 1 - par)

                @pl.when((blk > 0) | (p > 0))
                def _():
                    wait_out(par)
                compute(par)
                start_out(h * TB + tp, bb, par)
            return c

        lax.fori_loop(0, TB // 2, pairstep, 0)
        return carry

    lax.fori_loop(0, 4, blkstep, 0)
    wait_out(0)
    wait_out(1)


_gather_mean = functools.partial(
    pl.kernel,
    out_type=jax.ShapeDtypeStruct((NTOK, B, D), jnp.float32),
    mesh=plsc.VectorSubcoreMesh(core_axis_name="c", subcore_axis_name="s"),
    scratch_types=[
        pltpu.VMEM((TB, SUB * BB), jnp.int32),
        pltpu.VMEM((SUB * BB, CW), jnp.float32),
        pltpu.VMEM((SUB * BB, CW), jnp.float32),
        pltpu.VMEM((BB, CW), jnp.float32),
        pltpu.VMEM((BB, CW), jnp.float32),
        pltpu.SemaphoreType.DMA,
        pltpu.SemaphoreType.DMA,
        pltpu.SemaphoreType.DMA,
        pltpu.SemaphoreType.DMA,
    ],
)(_body)


def kernel(tokens_ids, seg_ids, emb_table):
    idx = (tokens_ids.reshape(NBB, BB, NTOK, SUB)
           .transpose(0, 2, 3, 1)
           .reshape(NBB, 2, TB, SUB * BB))
    out = _gather_mean(idx, emb_table)
    return out.transpose(1, 0, 2)


# speedup vs baseline: 6.1109x; 1.0386x over previous
import functools

import jax
import jax.numpy as jnp
from jax import lax
from jax.experimental import pallas as pl
from jax.experimental.pallas import tpu as pltpu
from jax.experimental.pallas import tpu_sc as plsc

B = 1024
L = 200
D = 1024
SUB = 2
NTOK = L // SUB
NCG = 4
NBW = 8
CW = D // NCG
BB = 64
NBB = B // BB
TB = 50
LANES = 16
VPT = CW // LANES


def _body(idx_hbm, table_hbm, out_hbm,
          idx_v, rows0, rows1, out0, out1, sg0, sg1, so0, so1):
    wid = lax.axis_index("s") * 2 + lax.axis_index("c")
    gc = wid % NCG
    bw = wid // NCG
    col = pl.multiple_of(gc * CW, CW)
    rows = (rows0, rows1)
    outs = (out0, out1)
    sgs = (sg0, sg1)
    sos = (so0, so1)

    def start_gather(tp, par):
        for j in range(SUB):
            pltpu.async_copy(
                table_hbm.at[idx_v.at[tp, j], pl.ds(col, CW)],
                rows[par].at[pl.ds(j * BB, BB)], sgs[par])

    def wait_gather(par):
        for j in range(SUB):
            pltpu.make_async_copy(
                table_hbm.at[idx_v.at[0, 0], pl.ds(col, CW)],
                rows[par].at[pl.ds(0, BB)], sgs[par]).wait()

    def start_out(t, bb, par):
        b0 = pl.multiple_of(bb * BB, BB)
        pltpu.async_copy(
            outs[par], out_hbm.at[t, pl.ds(b0, BB), pl.ds(col, CW)], sos[par])

    def wait_out(par):
        pltpu.make_async_copy(
            outs[par], out_hbm.at[0, pl.ds(0, BB), pl.ds(col, CW)],
            sos[par]).wait()

    def compute(par):
        rv, ov = rows[par], outs[par]

        @plsc.parallel_loop(0, BB, unroll=2)
        def _(q):
            for j in range(VPT):
                s = pl.ds(j * LANES, LANES)
                ov[q, s] = (rv[q, s] + rv[BB + q, s]) * 0.5

    def blkstep(blk, carry):
        bb = bw * 2 + (blk >> 1)
        h = blk & 1
        pltpu.sync_copy(idx_hbm.at[bb, h], idx_v)
        start_gather(0, 0)

        def pairstep(p, c):
            for half in range(2):
                tp = 2 * p + half
                par = half
                wait_gather(par)
                if half == 0:
                    start_gather(tp + 1, 1 - par)
                else:
                    @pl.when(p < TB // 2 - 1)
                    def _():
                        start_gather(tp + 1, 1 - par)

                @pl.when((blk > 0) | (p > 0))
                def _():
                    wait_out(par)
                compute(par)
                start_out(h * TB + tp, bb, par)
            return c

        lax.fori_loop(0, TB // 2, pairstep, 0)
        return carry

    lax.fori_loop(0, 4, blkstep, 0)
    wait_out(0)
    wait_out(1)


_gather_mean = functools.partial(
    pl.kernel,
    out_type=jax.ShapeDtypeStruct((NTOK, B, D), jnp.float32),
    mesh=plsc.VectorSubcoreMesh(core_axis_name="c", subcore_axis_name="s"),
    scratch_types=[
        pltpu.VMEM((TB, SUB, BB), jnp.int32),
        pltpu.VMEM((SUB * BB, CW), jnp.float32),
        pltpu.VMEM((SUB * BB, CW), jnp.float32),
        pltpu.VMEM((BB, CW), jnp.float32),
        pltpu.VMEM((BB, CW), jnp.float32),
        pltpu.SemaphoreType.DMA,
        pltpu.SemaphoreType.DMA,
        pltpu.SemaphoreType.DMA,
        pltpu.SemaphoreType.DMA,
    ],
)(_body)


def kernel(tokens_ids, seg_ids, emb_table):
    idx = (tokens_ids.reshape(NBB, BB, NTOK, SUB)
           .transpose(0, 2, 3, 1)
           .reshape(NBB, 2, TB, SUB, BB))
    out = _gather_mean(idx, emb_table)
    return out.transpose(1, 0, 2)
